# baseline (speedup 1.0000x reference)
import functools

import jax
import jax.numpy as jnp
from jax import lax
from jax.experimental import pallas as pl
from jax.experimental.pallas import tpu as pltpu
from jax.experimental.pallas import tpu_sc as plsc

_N = 10000
_NP = 10240
_HN = _NP // 2
_ACC_ROWS = _HN + 128
_F = 64
_ALPHA = 0.1
_ITERS = 10

_TILES = 16
_GRP = 128
_GQ = 2
_QROWS = _GQ * _GRP
_SUP = 4 * _QROWS
_GPC = _SUP // _GRP
_NCHUNK = 21
_EPT = _SUP * _NCHUNK
_EPAD = _TILES * _EPT
_BROWS = 64


def _mlp_body(f_ref, w1_ref, b1_ref, w2_ref, b2_ref, ws_ref,
              s0_ref, u_ref, v_ref, uf_ref, vf_ref):
    h = jnp.maximum(jnp.dot(f_ref[...], w1_ref[...],
                            preferred_element_type=jnp.float32) + b1_ref[...], 0.0)
    z = jnp.dot(h, w2_ref[...], preferred_element_type=jnp.float32) + b2_ref[...]
    ws = ws_ref[...]
    dinv = jnp.sqrt(ws)
    s0 = dinv * z
    s0_ref[...] = s0
    v_ref[...] = _ALPHA * s0
    u_ref[...] = jnp.broadcast_to((1.0 - _ALPHA) * ws, z.shape)
    uf_ref[...] = jnp.broadcast_to((1.0 - _ALPHA) * dinv, z.shape)
    vf_ref[...] = _ALPHA * z


def _mlp(features, W1, b1, W2, b2, ws):
    bs = 1000
    grid = (_N // bs,)
    outs = [jax.ShapeDtypeStruct((_N, _F), jnp.float32)] * 5
    return pl.pallas_call(
        _mlp_body,
        grid=grid,
        in_specs=[
            pl.BlockSpec((bs, 128), lambda i: (i, 0)),
            pl.BlockSpec((128, _F), lambda i: (0, 0)),
            pl.BlockSpec((1, _F), lambda i: (0, 0)),
            pl.BlockSpec((_F, _F), lambda i: (0, 0)),
            pl.BlockSpec((1, _F), lambda i: (0, 0)),
            pl.BlockSpec((bs, 1), lambda i: (i, 0)),
        ],
        out_specs=[pl.BlockSpec((bs, _F), lambda i: (i, 0))] * 5,
        out_shape=outs,
    )(features, W1, b1, W2, b2, ws)


_MESH = plsc.VectorSubcoreMesh(core_axis_name="c", subcore_axis_name="s")


@functools.partial(
    pl.kernel,
    out_type=jax.ShapeDtypeStruct((_NP, _F), jnp.float32),
    mesh=_MESH,
    compiler_params=pltpu.CompilerParams(use_tc_tiling_on_sc=False),
    scratch_types=[
        pltpu.VMEM_SHARED((_ACC_ROWS, _F), jnp.float32),
        pltpu.VMEM((_QROWS, _F), jnp.float32),
        pltpu.VMEM((_QROWS, _F), jnp.float32),
        pltpu.VMEM((_QROWS, _F), jnp.float32),
        pltpu.VMEM((_QROWS, _F), jnp.float32),
        pltpu.VMEM((2 * _GPC, _GRP), jnp.int32),
        pltpu.VMEM((2 * _GPC, _GRP), jnp.int32),
        pltpu.VMEM((_BROWS, _F), jnp.float32),
        pltpu.VMEM((_BROWS, _F), jnp.float32),
        pltpu.VMEM((_BROWS, _F), jnp.float32),
        pltpu.VMEM((_BROWS, _F), jnp.float32),
        pltpu.SemaphoreType.DMA,
        pltpu.SemaphoreType.DMA,
        pltpu.SemaphoreType.DMA,
        pltpu.SemaphoreType.DMA,
        pltpu.SemaphoreType.DMA,
        pltpu.SemaphoreType.DMA,
        pltpu.SemaphoreType.DMA,
        pltpu.SemaphoreType.DMA,
        pltpu.SemaphoreType.DMA,
    ],
)
def _prop_iter(src_h, u_h, v_h, rowc0_h, rowc1_h, col_h, dst_h,
               acc_sh, gbufa, gbufb, gbufc, gbufd, colbuf, rowbuf,
               abuf, ubuf, vbuf, zbuf,
               gsema, gsemb, gsemc, gsemd, ssema, ssemb, ssemc, ssemd, isem):
    cid = lax.axis_index("c")
    sid = lax.axis_index("s")
    gbufs = [gbufa, gbufb, gbufc, gbufd]
    gsems = [gsema, gsemb, gsemc, gsemd]
    ssems = [ssema, ssemb, ssemc, ssemd]

    def work(row_h):
        ebase = sid * (_EPT // _GRP)
        lbase = sid * (_HN // _TILES)
        hbase = cid * _HN + lbase

        def drain(buf, sem):
            pltpu.make_async_copy(src_h.at[pl.ds(0, _QROWS)], buf, sem).wait()

        def _zb(i, _):
            for q in range(4):
                zbuf[i, pl.ds(q * 16, 16)] = jnp.zeros((16,), jnp.float32)
            return 0
        lax.fori_loop(0, _BROWS, _zb, 0)

        zb = sid * (_ACC_ROWS // _TILES)
        def _init(b, _):
            pltpu.sync_copy(zbuf, acc_sh.at[pl.ds(zb + b * _BROWS, _BROWS)])
            return 0
        lax.fori_loop(0, 5, _init, 0)
        pltpu.sync_copy(zbuf.at[pl.ds(0, 8)],
                        acc_sh.at[pl.ds(zb + 5 * _BROWS, 8)])

        plsc.subcore_barrier()

        pltpu.sync_copy(col_h.at[pl.ds(ebase, _GPC)], colbuf.at[pl.ds(0, _GPC)])
        pltpu.sync_copy(row_h.at[pl.ds(ebase, _GPC)], rowbuf.at[pl.ds(0, _GPC)])

        def chunk(c, _):
            slot = lax.rem(c, 2)
            nslot = lax.rem(c + 1, 2)

            @pl.when(c > 0)
            def _wi():
                pltpu.make_async_copy(col_h.at[pl.ds(ebase, _GPC)],
                                      colbuf.at[pl.ds(0, _GPC)], isem).wait()
                pltpu.make_async_copy(row_h.at[pl.ds(ebase, _GPC)],
                                      rowbuf.at[pl.ds(0, _GPC)], isem).wait()

            for q in range(4):
                @pl.when(c > 0)
                def _dsq(q=q):
                    drain(gbufs[q], ssems[q])
                for j in range(_GQ):
                    pltpu.async_copy(
                        src_h.at[colbuf.at[slot * _GPC + q * _GQ + j]],
                        gbufs[q].at[pl.ds(j * _GRP, _GRP)], gsems[q])

            @pl.when(c < _NCHUNK - 1)
            def _pf():
                gb = ebase + (c + 1) * _GPC
                pltpu.async_copy(col_h.at[pl.ds(gb, _GPC)],
                                 colbuf.at[pl.ds(nslot * _GPC, _GPC)], isem)
                pltpu.async_copy(row_h.at[pl.ds(gb, _GPC)],
                                 rowbuf.at[pl.ds(nslot * _GPC, _GPC)], isem)

            for q in range(4):
                drain(gbufs[q], gsems[q])
                for j in range(_GQ):
                    pltpu.async_copy(
                        gbufs[q].at[pl.ds(j * _GRP, _GRP)],
                        acc_sh.at[rowbuf.at[slot * _GPC + q * _GQ + j]],
                        ssems[q], add=True)
            return 0
        lax.fori_loop(0, _NCHUNK, chunk, 0)

        for q in range(4):
            drain(gbufs[q], ssems[q])

        plsc.subcore_barrier()

        def _blc(b, _):
            lb = lbase + b * _BROWS
            hb = hbase + b * _BROWS
            pltpu.sync_copy(acc_sh.at[pl.ds(lb, _BROWS)], abuf)
            pltpu.sync_copy(u_h.at[pl.ds(hb, _BROWS)], ubuf)
            pltpu.sync_copy(v_h.at[pl.ds(hb, _BROWS)], vbuf)

            def _bl(i, _):
                for q in range(4):
                    sl = pl.ds(q * 16, 16)
                    abuf[i, sl] = abuf[i, sl] * ubuf[i, sl] + vbuf[i, sl]
                return 0
            lax.fori_loop(0, _BROWS, _bl, 0)
            pltpu.sync_copy(abuf, dst_h.at[pl.ds(hb, _BROWS)])
            return 0
        lax.fori_loop(0, _HN // _TILES // _BROWS, _blc, 0)

    @pl.when(cid == 0)
    def _c0():
        work(rowc0_h)

    @pl.when(cid == 1)
    def _c1():
        work(rowc1_h)


def _pad_rows(x):
    return jnp.concatenate(
        [x, jnp.zeros((_NP - _N, _F), jnp.float32)], axis=0)


def _remap(row, core):
    t = row - core * _HN
    own = (t >= 0) & (t < _HN)
    return jnp.where(own, t, _HN + (row & 127)).astype(jnp.int32)


def kernel(features, edge_index, edge_weights, W1, b1, W2, b2):
    row = edge_index[0].astype(jnp.int32)
    col = edge_index[1].astype(jnp.int32)
    et = edge_weights.shape[0]
    ws = edge_weights[et - _N:].reshape(_N, 1)

    pad = _EPAD - et
    ar = jnp.arange(pad, dtype=jnp.int32)
    prow = _NP + (ar % 128)
    pcol = ar % _N
    rowp = jnp.concatenate([row, prow])
    rowc0 = _remap(rowp, 0).reshape(_EPAD // _GRP, _GRP)
    rowc1 = _remap(rowp, 1).reshape(_EPAD // _GRP, _GRP)
    col2 = jnp.concatenate([col, pcol]).reshape(_EPAD // _GRP, _GRP)

    s0, u, v, uf, vf = _mlp(features, W1, b1.reshape(1, _F), W2,
                            b2.reshape(1, _F), ws)
    s0, u, v, uf, vf = (_pad_rows(a) for a in (s0, u, v, uf, vf))
    s = s0
    for _ in range(_ITERS - 1):
        s = _prop_iter(s, u, v, rowc0, rowc1, col2)
    out = _prop_iter(s, uf, vf, rowc0, rowc1, col2)
    return out[:_N]


# speedup vs baseline: 1.0144x; 1.0144x over previous
import functools

import jax
import jax.numpy as jnp
from jax import lax
from jax.experimental import pallas as pl
from jax.experimental.pallas import tpu as pltpu
from jax.experimental.pallas import tpu_sc as plsc

_N = 10000
_NP = 10240
_HN = _NP // 2
_ACC_ROWS = _HN + 128
_F = 64
_ALPHA = 0.1
_ITERS = 10

_TILES = 16
_GRP = 128
_G4 = 4
_HALF = _G4 * _GRP
_SUP = 2 * _HALF
_GPC = _SUP // _GRP
_NCHUNK = 21
_EPT = _SUP * _NCHUNK
_EPAD = _TILES * _EPT
_BROWS = 64


def _mlp_body(f_ref, w1_ref, b1_ref, w2_ref, b2_ref, ws_ref,
              s0_ref, u_ref, v_ref, uf_ref, vf_ref):
    h = jnp.maximum(jnp.dot(f_ref[...], w1_ref[...],
                            preferred_element_type=jnp.float32) + b1_ref[...], 0.0)
    z = jnp.dot(h, w2_ref[...], preferred_element_type=jnp.float32) + b2_ref[...]
    ws = ws_ref[...]
    dinv = jnp.sqrt(ws)
    s0 = dinv * z
    s0_ref[...] = s0
    v_ref[...] = _ALPHA * s0
    u_ref[...] = jnp.broadcast_to((1.0 - _ALPHA) * ws, z.shape)
    uf_ref[...] = jnp.broadcast_to((1.0 - _ALPHA) * dinv, z.shape)
    vf_ref[...] = _ALPHA * z


def _mlp(features, W1, b1, W2, b2, ws):
    bs = 1000
    grid = (_N // bs,)
    outs = [jax.ShapeDtypeStruct((_N, _F), jnp.float32)] * 5
    return pl.pallas_call(
        _mlp_body,
        grid=grid,
        in_specs=[
            pl.BlockSpec((bs, 128), lambda i: (i, 0)),
            pl.BlockSpec((128, _F), lambda i: (0, 0)),
            pl.BlockSpec((1, _F), lambda i: (0, 0)),
            pl.BlockSpec((_F, _F), lambda i: (0, 0)),
            pl.BlockSpec((1, _F), lambda i: (0, 0)),
            pl.BlockSpec((bs, 1), lambda i: (i, 0)),
        ],
        out_specs=[pl.BlockSpec((bs, _F), lambda i: (i, 0))] * 5,
        out_shape=outs,
    )(features, W1, b1, W2, b2, ws)


_MESH = plsc.VectorSubcoreMesh(core_axis_name="c", subcore_axis_name="s")


@functools.partial(
    pl.kernel,
    out_type=jax.ShapeDtypeStruct((_NP, _F), jnp.float32),
    mesh=_MESH,
    compiler_params=pltpu.CompilerParams(use_tc_tiling_on_sc=False),
    scratch_types=[
        pltpu.VMEM_SHARED((_ACC_ROWS, _F), jnp.float32),
        pltpu.VMEM((_HALF, _F), jnp.float32),
        pltpu.VMEM((_HALF, _F), jnp.float32),
        pltpu.VMEM((2 * _SUP,), jnp.int32),
        pltpu.VMEM((2 * _GPC, _GRP), jnp.int32),
        pltpu.VMEM((_BROWS, _F), jnp.float32),
        pltpu.VMEM((_BROWS, _F), jnp.float32),
        pltpu.VMEM((_BROWS, _F), jnp.float32),
        pltpu.VMEM((_BROWS, _F), jnp.float32),
        pltpu.SemaphoreType.DMA,
        pltpu.SemaphoreType.DMA,
        pltpu.SemaphoreType.DMA,
        pltpu.SemaphoreType.DMA,
        pltpu.SemaphoreType.DMA,
    ],
)
def _prop_iter(src_h, u_h, v_h, rowc0_h, rowc1_h, col_h, dst_h,
               acc_sh, gbuf0, gbuf1, colfb, rowbuf, abuf, ubuf, vbuf, zbuf,
               gsem0, gsem1, ssem0, ssem1, isem):
    cid = lax.axis_index("c")
    sid = lax.axis_index("s")

    def work(row_h):
        ebase = sid * (_EPT // _GRP)
        ebase1 = sid * _EPT
        lbase = sid * (_HN // _TILES)
        hbase = cid * _HN + lbase

        def drain(buf, sem):
            pltpu.make_async_copy(src_h.at[pl.ds(0, _HALF)], buf, sem).wait()

        def _zb(i, _):
            for q in range(4):
                zbuf[i, pl.ds(q * 16, 16)] = jnp.zeros((16,), jnp.float32)
            return 0
        lax.fori_loop(0, _BROWS, _zb, 0)

        zb = sid * (_ACC_ROWS // _TILES)
        def _init(b, _):
            pltpu.sync_copy(zbuf, acc_sh.at[pl.ds(zb + b * _BROWS, _BROWS)])
            return 0
        lax.fori_loop(0, 5, _init, 0)
        pltpu.sync_copy(zbuf.at[pl.ds(0, 8)],
                        acc_sh.at[pl.ds(zb + 5 * _BROWS, 8)])

        plsc.subcore_barrier()

        pltpu.sync_copy(col_h.at[pl.ds(ebase1, _SUP)], colfb.at[pl.ds(0, _SUP)])
        pltpu.sync_copy(row_h.at[pl.ds(ebase, _GPC)], rowbuf.at[pl.ds(0, _GPC)])

        def chunk(c, _):
            slot = lax.rem(c, 2)
            nslot = lax.rem(c + 1, 2)

            @pl.when(c > 0)
            def _wi():
                pltpu.make_async_copy(col_h.at[pl.ds(ebase1, _SUP)],
                                      colfb.at[pl.ds(0, _SUP)], isem).wait()
                pltpu.make_async_copy(row_h.at[pl.ds(ebase, _GPC)],
                                      rowbuf.at[pl.ds(0, _GPC)], isem).wait()

            @pl.when(c > 0)
            def _ds0():
                drain(gbuf0, ssem0)
            pltpu.async_copy(
                src_h.at[colfb.at[pl.ds(slot * _SUP, _HALF)]], gbuf0, gsem0)

            @pl.when(c > 0)
            def _sg1():
                drain(gbuf1, gsem1)
                pslot = lax.rem(c + 1, 2)
                for j in range(_G4):
                    pltpu.async_copy(
                        gbuf1.at[pl.ds(j * _GRP, _GRP)],
                        acc_sh.at[rowbuf.at[pslot * _GPC + _G4 + j]],
                        ssem1, add=True)
                drain(gbuf1, ssem1)

            @pl.when(c < _NCHUNK - 1)
            def _pf():
                pltpu.async_copy(
                    col_h.at[pl.ds(ebase1 + (c + 1) * _SUP, _SUP)],
                    colfb.at[pl.ds(nslot * _SUP, _SUP)], isem)
                pltpu.async_copy(
                    row_h.at[pl.ds(ebase + (c + 1) * _GPC, _GPC)],
                    rowbuf.at[pl.ds(nslot * _GPC, _GPC)], isem)

            pltpu.async_copy(
                src_h.at[colfb.at[pl.ds(slot * _SUP + _HALF, _HALF)]],
                gbuf1, gsem1)

            drain(gbuf0, gsem0)
            for j in range(_G4):
                pltpu.async_copy(gbuf0.at[pl.ds(j * _GRP, _GRP)],
                                 acc_sh.at[rowbuf.at[slot * _GPC + j]],
                                 ssem0, add=True)
            return 0
        lax.fori_loop(0, _NCHUNK, chunk, 0)

        drain(gbuf1, gsem1)
        lslot = (_NCHUNK - 1) % 2
        for j in range(_G4):
            pltpu.async_copy(gbuf1.at[pl.ds(j * _GRP, _GRP)],
                             acc_sh.at[rowbuf.at[lslot * _GPC + _G4 + j]],
                             ssem1, add=True)
        drain(gbuf0, ssem0)
        drain(gbuf1, ssem1)

        plsc.subcore_barrier()

        def _blc(b, _):
            lb = lbase + b * _BROWS
            hb = hbase + b * _BROWS
            pltpu.sync_copy(acc_sh.at[pl.ds(lb, _BROWS)], abuf)
            pltpu.sync_copy(u_h.at[pl.ds(hb, _BROWS)], ubuf)
            pltpu.sync_copy(v_h.at[pl.ds(hb, _BROWS)], vbuf)

            def _bl(i, _):
                for q in range(4):
                    sl = pl.ds(q * 16, 16)
                    abuf[i, sl] = abuf[i, sl] * ubuf[i, sl] + vbuf[i, sl]
                return 0
            lax.fori_loop(0, _BROWS, _bl, 0)
            pltpu.sync_copy(abuf, dst_h.at[pl.ds(hb, _BROWS)])
            return 0
        lax.fori_loop(0, _HN // _TILES // _BROWS, _blc, 0)

    @pl.when(cid == 0)
    def _c0():
        work(rowc0_h)

    @pl.when(cid == 1)
    def _c1():
        work(rowc1_h)


def _pad_rows(x):
    return jnp.concatenate(
        [x, jnp.zeros((_NP - _N, _F), jnp.float32)], axis=0)


def _remap(row, core):
    t = row - core * _HN
    own = (t >= 0) & (t < _HN)
    return jnp.where(own, t, _HN + (row & 127)).astype(jnp.int32)


def kernel(features, edge_index, edge_weights, W1, b1, W2, b2):
    row = edge_index[0].astype(jnp.int32)
    col = edge_index[1].astype(jnp.int32)
    et = edge_weights.shape[0]
    ws = edge_weights[et - _N:].reshape(_N, 1)

    pad = _EPAD - et
    ar = jnp.arange(pad, dtype=jnp.int32)
    prow = _NP + (ar % 128)
    pcol = ar % _N
    rowp = jnp.concatenate([row, prow])
    rowc0 = _remap(rowp, 0).reshape(_EPAD // _GRP, _GRP)
    rowc1 = _remap(rowp, 1).reshape(_EPAD // _GRP, _GRP)
    col2 = jnp.concatenate([col, pcol])

    s0, u, v, uf, vf = _mlp(features, W1, b1.reshape(1, _F), W2,
                            b2.reshape(1, _F), ws)
    s0, u, v, uf, vf = (_pad_rows(a) for a in (s0, u, v, uf, vf))
    s = s0
    for _ in range(_ITERS - 1):
        s = _prop_iter(s, u, v, rowc0, rowc1, col2)
    out = _prop_iter(s, uf, vf, rowc0, rowc1, col2)
    return out[:_N]


# speedup vs baseline: 1.0658x; 1.0507x over previous
import functools

import jax
import jax.numpy as jnp
from jax import lax
from jax.experimental import pallas as pl
from jax.experimental.pallas import tpu as pltpu
from jax.experimental.pallas import tpu_sc as plsc

_N = 10000
_NP = 10240
_ACC_ROWS = _NP + 128
_F = 64
_ALPHA = 0.1
_ITERS = 10

_TILES = 16
_GRP = 128
_G4 = 4
_HALF = _G4 * _GRP
_SUP = 2 * _HALF
_GPC = _SUP // _GRP
_NCHUNK = 11
_EPT = _SUP * _NCHUNK
_EPC = _TILES * _EPT
_EPAD = 2 * _EPC
_BROWS = 64


def _mlp_body(f_ref, w1_ref, b1_ref, w2_ref, b2_ref, ws_ref,
              p0_ref, u_ref, v_ref, uf_ref, vf_ref):
    h = jnp.maximum(jnp.dot(f_ref[...], w1_ref[...],
                            preferred_element_type=jnp.float32) + b1_ref[...], 0.0)
    z = jnp.dot(h, w2_ref[...], preferred_element_type=jnp.float32) + b2_ref[...]
    ws = ws_ref[...]
    dinv = jnp.sqrt(ws)
    s0 = dinv * z
    p0_ref[...] = 0.5 * z / dinv
    v_ref[...] = _ALPHA * s0
    u_ref[...] = jnp.broadcast_to((1.0 - _ALPHA) * ws, z.shape)
    uf_ref[...] = jnp.broadcast_to((1.0 - _ALPHA) * dinv, z.shape)
    vf_ref[...] = _ALPHA * z


def _mlp(features, W1, b1, W2, b2, ws):
    bs = 1000
    grid = (_N // bs,)
    outs = [jax.ShapeDtypeStruct((_N, _F), jnp.float32)] * 5
    return pl.pallas_call(
        _mlp_body,
        grid=grid,
        in_specs=[
            pl.BlockSpec((bs, 128), lambda i: (i, 0)),
            pl.BlockSpec((128, _F), lambda i: (0, 0)),
            pl.BlockSpec((1, _F), lambda i: (0, 0)),
            pl.BlockSpec((_F, _F), lambda i: (0, 0)),
            pl.BlockSpec((1, _F), lambda i: (0, 0)),
            pl.BlockSpec((bs, 1), lambda i: (i, 0)),
        ],
        out_specs=[pl.BlockSpec((bs, _F), lambda i: (i, 0))] * 5,
        out_shape=outs,
    )(features, W1, b1, W2, b2, ws)


def _final_body(pa_ref, pb_ref, uf_ref, vf_ref, o_ref):
    o_ref[...] = uf_ref[...] * (pa_ref[...] + pb_ref[...]) + vf_ref[...]


def _final_blend(pa, pb, uf, vf):
    bs = 1024
    grid = (_NP // bs,)
    return pl.pallas_call(
        _final_body,
        grid=grid,
        in_specs=[pl.BlockSpec((bs, _F), lambda i: (i, 0))] * 4,
        out_specs=pl.BlockSpec((bs, _F), lambda i: (i, 0)),
        out_shape=jax.ShapeDtypeStruct((_NP, _F), jnp.float32),
    )(pa[:_NP], pb[:_NP], uf, vf)


_MESH = plsc.VectorSubcoreMesh(core_axis_name="c", subcore_axis_name="s")


@functools.partial(
    pl.kernel,
    out_type=(
        jax.ShapeDtypeStruct((_NP, _F), jnp.float32),
        jax.ShapeDtypeStruct((_NP, _F), jnp.float32),
        jax.ShapeDtypeStruct((_ACC_ROWS, _F), jnp.float32),
        jax.ShapeDtypeStruct((_ACC_ROWS, _F), jnp.float32),
    ),
    mesh=_MESH,
    compiler_params=pltpu.CompilerParams(use_tc_tiling_on_sc=False),
    scratch_types=[
        pltpu.VMEM_SHARED((_ACC_ROWS, _F), jnp.float32),
        pltpu.VMEM((_HALF, _F), jnp.float32),
        pltpu.VMEM((_HALF, _F), jnp.float32),
        pltpu.VMEM((2 * _SUP,), jnp.int32),
        pltpu.VMEM((2 * _GPC, _GRP), jnp.int32),
        pltpu.VMEM((_BROWS, _F), jnp.float32),
        pltpu.VMEM((_BROWS, _F), jnp.float32),
        pltpu.VMEM((_BROWS, _F), jnp.float32),
        pltpu.VMEM((_BROWS, _F), jnp.float32),
        pltpu.SemaphoreType.DMA,
        pltpu.SemaphoreType.DMA,
        pltpu.SemaphoreType.DMA,
        pltpu.SemaphoreType.DMA,
        pltpu.SemaphoreType.DMA,
    ],
)
def _prop_iter(pa_h, pb_h, u_h, v_h, row_h, col_h, zeros_h,
               sc0_h, sc1_h, pa2_h, pb2_h,
               acc_sh, gbuf0, gbuf1, colfb, rowbuf, abuf, bbuf, ubuf, vbuf,
               gsem0, gsem1, ssem0, ssem1, isem):
    cid = lax.axis_index("c")
    sid = lax.axis_index("s")

    def work(s_h, pout_h):
        ebase = cid * (_EPC // _GRP) + sid * (_EPT // _GRP)
        ebase1 = cid * _EPC + sid * _EPT
        rbase = sid * (_NP // _TILES)
        abase = sid * (_ACC_ROWS // _TILES)

        def drain(buf, sem):
            pltpu.make_async_copy(s_h.at[pl.ds(0, _HALF)], buf, sem).wait()

        def _blc(b, _):
            rb = rbase + b * _BROWS
            pltpu.sync_copy(pa_h.at[pl.ds(rb, _BROWS)], abuf)
            pltpu.sync_copy(pb_h.at[pl.ds(rb, _BROWS)], bbuf)
            pltpu.sync_copy(u_h.at[pl.ds(rb, _BROWS)], ubuf)
            pltpu.sync_copy(v_h.at[pl.ds(rb, _BROWS)], vbuf)

            def _bl(i, _):
                for q in range(4):
                    sl = pl.ds(q * 16, 16)
                    abuf[i, sl] = (abuf[i, sl] + bbuf[i, sl]) * ubuf[i, sl] \
                        + vbuf[i, sl]
                return 0
            lax.fori_loop(0, _BROWS, _bl, 0)
            pltpu.sync_copy(abuf, s_h.at[pl.ds(rb, _BROWS)])
            pltpu.sync_copy(zeros_h, acc_sh.at[pl.ds(abase + b * _BROWS, _BROWS)])
            return 0
        lax.fori_loop(0, _NP // _TILES // _BROWS, _blc, 0)
        pltpu.sync_copy(zeros_h.at[pl.ds(0, 8)],
                        acc_sh.at[pl.ds(abase + 640, 8)])

        plsc.subcore_barrier()

        pltpu.sync_copy(col_h.at[pl.ds(ebase1, _SUP)], colfb.at[pl.ds(0, _SUP)])
        pltpu.sync_copy(row_h.at[pl.ds(ebase, _GPC)], rowbuf.at[pl.ds(0, _GPC)])

        def chunk(c, _):
            slot = lax.rem(c, 2)
            nslot = lax.rem(c + 1, 2)

            @pl.when(c > 0)
            def _wi():
                pltpu.make_async_copy(col_h.at[pl.ds(ebase1, _SUP)],
                                      colfb.at[pl.ds(0, _SUP)], isem).wait()
                pltpu.make_async_copy(row_h.at[pl.ds(ebase, _GPC)],
                                      rowbuf.at[pl.ds(0, _GPC)], isem).wait()

            @pl.when(c > 0)
            def _ds0():
                drain(gbuf0, ssem0)
            pltpu.async_copy(
                s_h.at[colfb.at[pl.ds(slot * _SUP, _HALF)]], gbuf0, gsem0)

            @pl.when(c > 0)
            def _sg1():
                drain(gbuf1, gsem1)
                pslot = lax.rem(c + 1, 2)
                for j in range(_G4):
                    pltpu.async_copy(
                        gbuf1.at[pl.ds(j * _GRP, _GRP)],
                        acc_sh.at[rowbuf.at[pslot * _GPC + _G4 + j]],
                        ssem1, add=True)
                drain(gbuf1, ssem1)

            @pl.when(c < _NCHUNK - 1)
            def _pf():
                pltpu.async_copy(
                    col_h.at[pl.ds(ebase1 + (c + 1) * _SUP, _SUP)],
                    colfb.at[pl.ds(nslot * _SUP, _SUP)], isem)
                pltpu.async_copy(
                    row_h.at[pl.ds(ebase + (c + 1) * _GPC, _GPC)],
                    rowbuf.at[pl.ds(nslot * _GPC, _GPC)], isem)

            pltpu.async_copy(
                s_h.at[colfb.at[pl.ds(slot * _SUP + _HALF, _HALF)]],
                gbuf1, gsem1)

            drain(gbuf0, gsem0)
            for j in range(_G4):
                pltpu.async_copy(gbuf0.at[pl.ds(j * _GRP, _GRP)],
                                 acc_sh.at[rowbuf.at[slot * _GPC + j]],
                                 ssem0, add=True)
            return 0
        lax.fori_loop(0, _NCHUNK, chunk, 0)

        drain(gbuf1, gsem1)
        lslot = (_NCHUNK - 1) % 2
        for j in range(_G4):
            pltpu.async_copy(gbuf1.at[pl.ds(j * _GRP, _GRP)],
                             acc_sh.at[rowbuf.at[lslot * _GPC + _G4 + j]],
                             ssem1, add=True)
        drain(gbuf0, ssem0)
        drain(gbuf1, ssem1)

        plsc.subcore_barrier()

        pltpu.sync_copy(acc_sh.at[pl.ds(abase, _ACC_ROWS // _TILES)],
                        pout_h.at[pl.ds(abase, _ACC_ROWS // _TILES)])

    @pl.when(cid == 0)
    def _c0():
        work(sc0_h, pa2_h)

    @pl.when(cid == 1)
    def _c1():
        work(sc1_h, pb2_h)


def _pad_rows(x):
    return jnp.concatenate(
        [x, jnp.zeros((_NP - _N, _F), jnp.float32)], axis=0)


def kernel(features, edge_index, edge_weights, W1, b1, W2, b2):
    row = edge_index[0].astype(jnp.int32)
    col = edge_index[1].astype(jnp.int32)
    et = edge_weights.shape[0]
    ws = edge_weights[et - _N:].reshape(_N, 1)

    pad = _EPAD - et
    ar = jnp.arange(pad, dtype=jnp.int32)
    prow = _NP + (ar % 128)
    pcol = ar % _N
    row2 = jnp.concatenate([row, prow]).reshape(_EPAD // _GRP, _GRP)
    col2 = jnp.concatenate([col, pcol])
    zeros = jnp.zeros((_BROWS, _F), jnp.float32)

    p0, u, v, uf, vf = _mlp(features, W1, b1.reshape(1, _F), W2,
                            b2.reshape(1, _F), ws)
    u, v, uf, vf = (_pad_rows(a) for a in (u, v, uf, vf))
    pa = jnp.concatenate(
        [p0, jnp.zeros((_ACC_ROWS - _N, _F), jnp.float32)], axis=0)
    pb = pa
    for _ in range(_ITERS):
        _, _, pa, pb = _prop_iter(pa, pb, u, v, row2, col2, zeros)
    out = _final_blend(pa, pb, uf, vf)
    return out[:_N]
